# baseline (speedup 1.0000x reference)
import jax
import jax.numpy as jnp
from jax import lax
from jax.experimental import pallas as pl
from jax.experimental.pallas import tpu as pltpu
from jax.experimental.pallas import tpu_sc as plsc

DX = (0.5, 0.5)
BX = (-49.75, -49.75)
BEV_DIM = (200, 200)

B = 256
T = 1800
L = 16
NC = 2
NS = 16
NW = NC * NS
BPW = B // NW
NCHUNK = (T + L - 1) // L
TPAD = NCHUNK * L


def _sc_body(trajs_hbm, c_hbm, out_hbm, c_v0, c_v1, tr_v, out_v, sem0, sem1):
    wid = lax.axis_index("s") * NC + lax.axis_index("c")
    base = wid * BPW
    c_bufs = (c_v0, c_v1)
    sems = (sem0, sem1)

    inv_dx = float(1.0 / DX[0])
    neg_bx = float(-BX[0])

    pltpu.make_async_copy(c_hbm.at[base], c_bufs[0], sems[0]).start()

    for k in range(BPW):
        b = base + k
        c_v = c_bufs[k % 2]
        pltpu.make_async_copy(c_hbm.at[b], c_v, sems[k % 2]).wait()
        if k + 1 < BPW:
            pltpu.make_async_copy(
                c_hbm.at[b + 1], c_bufs[(k + 1) % 2], sems[(k + 1) % 2]
            ).start()
        pltpu.sync_copy(trajs_hbm.at[pl.ds(b * (2 * T), 2 * T)], tr_v)

        def chunk(i, _, c_v=c_v):
            t = jnp.minimum(i * L + lax.iota(jnp.int32, L), T - 1)
            xx = plsc.load_gather(tr_v, [t * 2])
            yy = plsc.load_gather(tr_v, [t * 2 + 1])
            xi = jnp.clip(((xx + neg_bx) * inv_dx).astype(jnp.int32),
                          0, BEV_DIM[0] - 1)
            yi = jnp.clip(((yy + neg_bx) * inv_dx).astype(jnp.int32),
                          0, BEV_DIM[1] - 1)
            out_v[pl.ds(i * L, L)] = plsc.load_gather(c_v, [xi * BEV_DIM[1] + yi])
            return 0

        lax.fori_loop(0, NCHUNK, chunk, 0)
        pltpu.sync_copy(out_v.at[pl.ds(0, T)], out_hbm.at[pl.ds(b * T, T)])


@jax.jit
def kernel(trajs, C):
    trajs_flat = trajs.reshape(B * T * 2)
    run = pl.kernel(
        _sc_body,
        out_type=jax.ShapeDtypeStruct((B * T,), jnp.float32),
        mesh=plsc.VectorSubcoreMesh(
            core_axis_name="c", subcore_axis_name="s",
            num_cores=NC, num_subcores=NS),
        scratch_types=[
            pltpu.VMEM((BEV_DIM[0] * BEV_DIM[1],), jnp.float32),
            pltpu.VMEM((BEV_DIM[0] * BEV_DIM[1],), jnp.float32),
            pltpu.VMEM((2 * T,), jnp.float32),
            pltpu.VMEM((TPAD,), jnp.float32),
            pltpu.SemaphoreType.DMA,
            pltpu.SemaphoreType.DMA,
        ],
        compiler_params=pltpu.CompilerParams(
            needs_layout_passes=False, use_tc_tiling_on_sc=False),
    )
    return run(trajs_flat, C.reshape(B, BEV_DIM[0] * BEV_DIM[1])).reshape(B, T)


# speedup vs baseline: 3.2568x; 3.2568x over previous
import jax
import jax.numpy as jnp
from jax import lax
from jax.experimental import pallas as pl
from jax.experimental.pallas import tpu as pltpu
from jax.experimental.pallas import tpu_sc as plsc

DX = (0.5, 0.5)
BX = (-49.75, -49.75)
BEV_DIM = (200, 200)

B = 256
T = 1800
L = 16
NC = 2
NS = 16
NW = NC * NS
BPW = B // NW
NFULL = T // L


def _sc_body(xs_hbm, ys_hbm, c_hbm, out_hbm, xs_v, ys_v, c_v, out_v, sem):
    wid = lax.axis_index("s") * NC + lax.axis_index("c")
    base = wid * BPW

    inv_dx = float(1.0 / DX[0])
    neg_bx = float(-BX[0])

    pltpu.sync_copy(xs_hbm.at[wid], xs_v)
    pltpu.sync_copy(ys_hbm.at[wid], ys_v)

    for k in range(BPW):
        pltpu.sync_copy(c_hbm.at[base + k], c_v)

        def discretize(xx, yy):
            xi = jnp.clip(((xx + neg_bx) * inv_dx).astype(jnp.int32),
                          0, BEV_DIM[0] - 1)
            yi = jnp.clip(((yy + neg_bx) * inv_dx).astype(jnp.int32),
                          0, BEV_DIM[1] - 1)
            return xi, yi

        def chunk(i, _):
            c0 = i * L
            xi, yi = discretize(xs_v[k, pl.ds(c0, L)], ys_v[k, pl.ds(c0, L)])
            out_v[k, pl.ds(c0, L)] = plsc.load_gather(c_v, [xi, yi])
            return 0

        lax.fori_loop(0, NFULL, chunk, 0)

        kvec = jnp.full((L,), k, dtype=jnp.int32)
        tcol = jnp.minimum(NFULL * L + lax.iota(jnp.int32, L), T - 1)
        xi, yi = discretize(plsc.load_gather(xs_v, [kvec, tcol]),
                            plsc.load_gather(ys_v, [kvec, tcol]))
        plsc.store_scatter(out_v, [kvec, tcol],
                           plsc.load_gather(c_v, [xi, yi]))

    pltpu.sync_copy(out_v, out_hbm.at[wid])


@jax.jit
def kernel(trajs, C):
    xs = trajs[:, :, 0].reshape(NW, BPW, T)
    ys = trajs[:, :, 1].reshape(NW, BPW, T)
    run = pl.kernel(
        _sc_body,
        out_type=jax.ShapeDtypeStruct((NW, BPW, T), jnp.float32),
        mesh=plsc.VectorSubcoreMesh(
            core_axis_name="c", subcore_axis_name="s",
            num_cores=NC, num_subcores=NS),
        scratch_types=[
            pltpu.VMEM((BPW, T), jnp.float32),
            pltpu.VMEM((BPW, T), jnp.float32),
            pltpu.VMEM((BEV_DIM[0], BEV_DIM[1]), jnp.float32),
            pltpu.VMEM((BPW, T), jnp.float32),
            pltpu.SemaphoreType.DMA,
        ],
        compiler_params=pltpu.CompilerParams(
            needs_layout_passes=False, use_tc_tiling_on_sc=True),
    )
    return run(xs, ys, C).reshape(B, T)


# speedup vs baseline: 8.9126x; 2.7366x over previous
import jax
import jax.numpy as jnp
from jax import lax
from jax.experimental import pallas as pl
from jax.experimental.pallas import tpu as pltpu
from jax.experimental.pallas import tpu_sc as plsc

DX = (0.5, 0.5)
BX = (-49.75, -49.75)
BEV_DIM = (200, 200)

B = 256
T = 1800
L = 16
NC = 2
NS = 16
HB = 128
NSLAB = 16
TSTEP = 112
TLEN = 120
NG = HB // L
W0 = 88
WW = 24


def _sc_body(tr_hbm, c_hbm, out_hbm, cc_v, trx_v, try_v, out_v,
             ix_v, iy_v, row_v, sem_c, sem_x, sem_y):
    wid = lax.axis_index("s") * NC + lax.axis_index("c")
    slab = wid % NSLAB
    half = wid // NSLAB
    t0 = slab * TSTEP
    col0 = half * HB

    inv_dx = float(1.0 / DX[0])
    neg_bx = float(-BX[0])
    iota = lax.iota(jnp.int32, L)

    ccopy = pltpu.make_async_copy(
        c_hbm.at[pl.ds(W0, WW), pl.ds(W0, WW), pl.ds(col0, HB)], cc_v, sem_c)
    ccopy.start()

    for j in range(NG):
        base = 16 * j if j < NG - 1 else TLEN - L
        tvec = t0 + base + iota
        ix_v[pl.ds(base, L)] = 4 * tvec + 2 * half
        iy_v[pl.ds(base, L)] = 4 * tvec + 2 * half + 1
    xcopy = pltpu.make_async_copy(tr_hbm.at[ix_v], trx_v, sem_x)
    ycopy = pltpu.make_async_copy(tr_hbm.at[iy_v], try_v, sem_y)
    xcopy.start()
    ycopy.start()
    xcopy.wait()
    ycopy.wait()
    ccopy.wait()

    def discretize(xx, yy):
        xi = jnp.clip(((xx + neg_bx) * inv_dx).astype(jnp.int32),
                      0, BEV_DIM[0] - 1)
        yi = jnp.clip(((yy + neg_bx) * inv_dx).astype(jnp.int32),
                      0, BEV_DIM[1] - 1)
        return xi, yi

    def step(tl, acc):
        for g in range(NG):
            xi, yi = discretize(trx_v[tl, pl.ds(g * L, L)],
                                try_v[tl, pl.ds(g * L, L)])
            xw = jnp.clip(xi - W0, 0, WW - 1)
            yw = jnp.clip(yi - W0, 0, WW - 1)
            out_v[tl, pl.ds(g * L, L)] = plsc.load_gather(
                cc_v, [xw, yw, g * L + iota])
            miss = jnp.where((xw != xi - W0) | (yw != yi - W0), 1, 0)
            acc = jnp.maximum(acc, miss)
        return acc

    acc = lax.fori_loop(0, TLEN, step, jnp.zeros((L,), jnp.int32))

    @pl.when(jnp.max(acc) > 0)
    def _fix():
        def fstep(tl, carry):
            for g in range(NG):
                xi, yi = discretize(trx_v[tl, pl.ds(g * L, L)],
                                    try_v[tl, pl.ds(g * L, L)])
                inwin = ((xi >= W0) & (xi < W0 + WW)
                         & (yi >= W0) & (yi < W0 + WW))
                miss = jnp.where(inwin, 0, 1)

                @pl.when(jnp.max(miss) > 0)
                def _group():
                    for l in range(L):
                        sel = jnp.where(iota == l, 1, 0)
                        m_l = jnp.sum(sel * miss)

                        @pl.when(m_l > 0)
                        def _lane():
                            xs = jnp.sum(sel * xi)
                            ys = jnp.sum(sel * yi)
                            pltpu.sync_copy(c_hbm.at[xs, ys], row_v)
                            val = plsc.load_gather(
                                row_v,
                                [jnp.full((L,), col0 + g * L + l, jnp.int32)])
                            plsc.store_scatter(
                                out_v,
                                [jnp.full((L,), tl, jnp.int32),
                                 jnp.full((L,), g * L + l, jnp.int32)],
                                val)
            return carry

        lax.fori_loop(0, TLEN, fstep, 0)

    pltpu.sync_copy(out_v, out_hbm.at[pl.ds(t0, TLEN), pl.ds(col0, HB)])


@jax.jit
def kernel(trajs, C):
    c3 = jnp.transpose(C, (1, 2, 0))
    tr = jnp.transpose(trajs.reshape(2, HB, T, 2), (2, 0, 3, 1))
    tr = tr.reshape(4 * T, HB)
    run = pl.kernel(
        _sc_body,
        out_type=jax.ShapeDtypeStruct((T, B), jnp.float32),
        mesh=plsc.VectorSubcoreMesh(
            core_axis_name="c", subcore_axis_name="s",
            num_cores=NC, num_subcores=NS),
        scratch_types=[
            pltpu.VMEM((WW, WW, HB), jnp.float32),
            pltpu.VMEM((TLEN, HB), jnp.float32),
            pltpu.VMEM((TLEN, HB), jnp.float32),
            pltpu.VMEM((TLEN, HB), jnp.float32),
            pltpu.VMEM((TLEN,), jnp.int32),
            pltpu.VMEM((TLEN,), jnp.int32),
            pltpu.VMEM((B,), jnp.float32),
            pltpu.SemaphoreType.DMA,
            pltpu.SemaphoreType.DMA,
            pltpu.SemaphoreType.DMA,
        ],
        compiler_params=pltpu.CompilerParams(
            needs_layout_passes=False, use_tc_tiling_on_sc=True),
    )
    return jnp.transpose(run(tr, c3))


# speedup vs baseline: 9.2992x; 1.0434x over previous
import jax
import jax.numpy as jnp
from jax import lax
from jax.experimental import pallas as pl
from jax.experimental.pallas import tpu as pltpu
from jax.experimental.pallas import tpu_sc as plsc

DX = (0.5, 0.5)
BX = (-49.75, -49.75)
BEV_DIM = (200, 200)

B = 256
T = 1800
L = 16
NC = 2
NS = 16
HB = 128
NSLAB = 16
TSTEP = 112
TLEN = 120
NG = HB // L
W0 = 88
WW = 24


def _sc_body(tr_hbm, c_hbm, out_hbm, cc_v, trx_v, try_v, out_v,
             ix_v, iy_v, row_v, sem_c, sem_x, sem_y):
    wid = lax.axis_index("s") * NC + lax.axis_index("c")
    slab = wid % NSLAB
    half = wid // NSLAB
    t0 = slab * TSTEP
    col0 = half * HB

    inv_dx = float(1.0 / DX[0])
    neg_bx = float(-BX[0])
    iota = lax.iota(jnp.int32, L)

    ccopy = pltpu.make_async_copy(
        c_hbm.at[pl.ds(W0, WW), pl.ds(W0, WW), pl.ds(col0, HB)], cc_v, sem_c)
    ccopy.start()

    for j in range(NG):
        base = 16 * j if j < NG - 1 else TLEN - L
        tvec = t0 + base + iota
        ix_v[pl.ds(base, L)] = 4 * tvec + 2 * half
        iy_v[pl.ds(base, L)] = 4 * tvec + 2 * half + 1
    xcopy = pltpu.make_async_copy(tr_hbm.at[ix_v], trx_v, sem_x)
    ycopy = pltpu.make_async_copy(tr_hbm.at[iy_v], try_v, sem_y)
    xcopy.start()
    ycopy.start()
    xcopy.wait()
    ycopy.wait()
    ccopy.wait()

    def discretize(xx, yy):
        xi = jnp.clip(((xx + neg_bx) * inv_dx).astype(jnp.int32),
                      0, BEV_DIM[0] - 1)
        yi = jnp.clip(((yy + neg_bx) * inv_dx).astype(jnp.int32),
                      0, BEV_DIM[1] - 1)
        return xi, yi

    wmax = jnp.uint32(WW - 1)

    def step(tl, acc):
        for g in range(NG):
            xx = trx_v[tl, pl.ds(g * L, L)]
            yy = try_v[tl, pl.ds(g * L, L)]
            xs = ((xx + neg_bx) * inv_dx).astype(jnp.int32) - W0
            ys = ((yy + neg_bx) * inv_dx).astype(jnp.int32) - W0
            xw = plsc.bitcast(
                jnp.minimum(plsc.bitcast(xs, jnp.uint32), wmax), jnp.int32)
            yw = plsc.bitcast(
                jnp.minimum(plsc.bitcast(ys, jnp.uint32), wmax), jnp.int32)
            out_v[tl, pl.ds(g * L, L)] = plsc.load_gather(
                cc_v, [xw, yw, g * L + iota])
            acc = acc | (xs ^ xw) | (ys ^ yw)
        return acc

    acc = lax.fori_loop(0, TLEN, step, jnp.zeros((L,), jnp.int32))

    @pl.when(jnp.max(jnp.where(acc != 0, 1, 0)) > 0)
    def _fix():
        def fstep(tl, carry):
            for g in range(NG):
                xi, yi = discretize(trx_v[tl, pl.ds(g * L, L)],
                                    try_v[tl, pl.ds(g * L, L)])
                inwin = ((xi >= W0) & (xi < W0 + WW)
                         & (yi >= W0) & (yi < W0 + WW))
                miss = jnp.where(inwin, 0, 1)

                @pl.when(jnp.max(miss) > 0)
                def _group():
                    for l in range(L):
                        sel = jnp.where(iota == l, 1, 0)
                        m_l = jnp.sum(sel * miss)

                        @pl.when(m_l > 0)
                        def _lane():
                            xs = jnp.sum(sel * xi)
                            ys = jnp.sum(sel * yi)
                            pltpu.sync_copy(c_hbm.at[xs, ys], row_v)
                            val = plsc.load_gather(
                                row_v,
                                [jnp.full((L,), col0 + g * L + l, jnp.int32)])
                            plsc.store_scatter(
                                out_v,
                                [jnp.full((L,), tl, jnp.int32),
                                 jnp.full((L,), g * L + l, jnp.int32)],
                                val)
            return carry

        lax.fori_loop(0, TLEN, fstep, 0)

    pltpu.sync_copy(out_v, out_hbm.at[pl.ds(t0, TLEN), pl.ds(col0, HB)])


@jax.jit
def kernel(trajs, C):
    c3 = jnp.transpose(C, (1, 2, 0))
    tr = jnp.transpose(trajs.reshape(2, HB, T, 2), (2, 0, 3, 1))
    tr = tr.reshape(4 * T, HB)
    run = pl.kernel(
        _sc_body,
        out_type=jax.ShapeDtypeStruct((T, B), jnp.float32),
        mesh=plsc.VectorSubcoreMesh(
            core_axis_name="c", subcore_axis_name="s",
            num_cores=NC, num_subcores=NS),
        scratch_types=[
            pltpu.VMEM((WW, WW, HB), jnp.float32),
            pltpu.VMEM((TLEN, HB), jnp.float32),
            pltpu.VMEM((TLEN, HB), jnp.float32),
            pltpu.VMEM((TLEN, HB), jnp.float32),
            pltpu.VMEM((TLEN,), jnp.int32),
            pltpu.VMEM((TLEN,), jnp.int32),
            pltpu.VMEM((B,), jnp.float32),
            pltpu.SemaphoreType.DMA,
            pltpu.SemaphoreType.DMA,
            pltpu.SemaphoreType.DMA,
        ],
        compiler_params=pltpu.CompilerParams(
            needs_layout_passes=False, use_tc_tiling_on_sc=True),
    )
    return jnp.transpose(run(tr, c3))
